# baseline (device time: 63671 ns/iter reference)
import jax
import jax.numpy as jnp
from jax import lax
from jax.experimental import pallas as pl
from jax.experimental.pallas import tpu as pltpu

N_DEV = 4
SQ = 1024
SKV = 1024
H_LOC = 8
DH = 128
D_LOC = H_LOC * DH
D_OUT = 1024
WINDOW = 128
SCALE = 0.08838834764831843
C = SQ // (2 * N_DEV)
HALF = SQ // 2
KB = 3 * WINDOW


def kernel(x, Wq, K_ext, V_ext, Wo):
    i = lax.axis_index("i")
    x2 = x.reshape(SQ, x.shape[-1]).astype(jnp.bfloat16)
    Wq_i = lax.dynamic_slice_in_dim(Wq, i * D_LOC, D_LOC, axis=1).astype(
        jnp.bfloat16
    )
    Wo_i = lax.dynamic_slice_in_dim(Wo, i * D_LOC, D_LOC, axis=0).astype(
        jnp.bfloat16
    )
    K2 = K_ext.reshape(SKV, D_LOC).astype(jnp.bfloat16)
    V2 = V_ext.reshape(SKV, D_LOC).astype(jnp.bfloat16)

    def body(
        x_ref,
        wq_ref,
        k_ref,
        v_ref,
        wo_ref,
        out_ref,
        acc_ref,
        send_cw,
        recv_cw,
        send_ccw,
        recv_ccw,
        cw_send_sems,
        cw_recv_sems,
        ccw_send_sems,
        ccw_recv_sems,
    ):
        my = lax.axis_index("i")
        left = lax.rem(my + (N_DEV - 1), N_DEV)
        right = lax.rem(my + 1, N_DEV)
        f32 = jnp.float32
        bf16 = jnp.bfloat16

        qi = lax.broadcasted_iota(jnp.int32, (C, KB), 0)
        ki = lax.broadcasted_iota(jnp.int32, (C, KB), 1)

        def compute_chunk(r0):
            r0 = pl.multiple_of(r0, C)
            kb = pl.multiple_of(jnp.clip(r0 - WINDOW, 0, SKV - KB), WINDOW)
            q = jnp.dot(
                x_ref[pl.ds(r0, C), :], wq_ref[...],
                preferred_element_type=f32,
            )
            mask = jnp.abs((qi + r0) - (ki + kb)) <= WINDOW
            ctxs = []
            for h in range(H_LOC):
                qh = q[:, h * DH : (h + 1) * DH].astype(bf16)
                kh = k_ref[pl.ds(kb, KB), h * DH : (h + 1) * DH]
                vh = v_ref[pl.ds(kb, KB), h * DH : (h + 1) * DH]
                s = (
                    lax.dot_general(
                        qh, kh, (((1,), (1,)), ((), ())),
                        preferred_element_type=f32,
                    )
                    * SCALE
                )
                s = jnp.where(mask, s, -1e9)
                m = jnp.max(s, axis=1, keepdims=True)
                w = jnp.exp(s - m)
                w = w / jnp.sum(w, axis=1, keepdims=True)
                ctxs.append(
                    jnp.dot(w.astype(bf16), vh, preferred_element_type=f32)
                )
            ctx = jnp.concatenate(ctxs, axis=1).astype(bf16)
            acc_ref[pl.ds(r0, C), :] = jnp.dot(
                ctx, wo_ref[...], preferred_element_type=f32
            )

        def cw_chunk(t):
            return lax.rem(my - t + 2 * N_DEV, N_DEV)

        def ccw_chunk(t):
            return lax.rem(my + t, N_DEV)

        def cw_rows(c):
            return pl.ds(pl.multiple_of(c * C, C), C)

        def ccw_rows(c):
            return pl.ds(pl.multiple_of(HALF + c * C, C), C)

        def start_rdma(src, dst, ssem, rsem, dev):
            r = pltpu.make_async_remote_copy(
                src_ref=src,
                dst_ref=dst,
                send_sem=ssem,
                recv_sem=rsem,
                device_id=(dev,),
                device_id_type=pl.DeviceIdType.MESH,
            )
            r.start()
            return r

        compute_chunk(cw_chunk(0) * C)
        compute_chunk(HALF + ccw_chunk(0) * C)

        barrier_sem = pltpu.get_barrier_semaphore()
        for nbr in [left, right]:
            pl.semaphore_signal(
                barrier_sem,
                inc=1,
                device_id=(nbr,),
                device_id_type=pl.DeviceIdType.MESH,
            )
        pl.semaphore_wait(barrier_sem, 2)

        send_cw[0] = acc_ref[cw_rows(cw_chunk(0)), :].astype(bf16)
        r_cw = start_rdma(
            send_cw.at[0], recv_cw.at[0],
            cw_send_sems.at[0], cw_recv_sems.at[0], right,
        )
        send_ccw[0] = acc_ref[ccw_rows(ccw_chunk(0)), :].astype(bf16)
        r_ccw = start_rdma(
            send_ccw.at[0], recv_ccw.at[0],
            ccw_send_sems.at[0], ccw_recv_sems.at[0], left,
        )

        for h in range(1, N_DEV - 1):
            compute_chunk(cw_chunk(h) * C)
            compute_chunk(HALF + ccw_chunk(h) * C)
            r_cw.wait()
            r_ccw.wait()
            send_cw[h] = (
                acc_ref[cw_rows(cw_chunk(h)), :] + recv_cw[h - 1].astype(f32)
            ).astype(bf16)
            r_cw = start_rdma(
                send_cw.at[h], recv_cw.at[h],
                cw_send_sems.at[h], cw_recv_sems.at[h], right,
            )
            send_ccw[h] = (
                acc_ref[ccw_rows(ccw_chunk(h)), :]
                + recv_ccw[h - 1].astype(f32)
            ).astype(bf16)
            r_ccw = start_rdma(
                send_ccw.at[h], recv_ccw.at[h],
                ccw_send_sems.at[h], ccw_recv_sems.at[h], left,
            )

        c_own_cw = lax.rem(my + 1, N_DEV)
        c_own_ccw = lax.rem(my + N_DEV - 1, N_DEV)
        compute_chunk(c_own_cw * C)
        compute_chunk(HALF + c_own_ccw * C)
        r_cw.wait()
        r_ccw.wait()

        red_cw = (
            acc_ref[cw_rows(c_own_cw), :]
            + recv_cw[N_DEV - 2].astype(f32)
        )
        send_cw[N_DEV - 1] = red_cw.astype(bf16)
        r_cw = start_rdma(
            send_cw.at[N_DEV - 1], recv_cw.at[N_DEV - 1],
            cw_send_sems.at[N_DEV - 1], cw_recv_sems.at[N_DEV - 1], right,
        )
        red_ccw = (
            acc_ref[ccw_rows(c_own_ccw), :]
            + recv_ccw[N_DEV - 2].astype(f32)
        )
        send_ccw[N_DEV - 1] = red_ccw.astype(bf16)
        r_ccw = start_rdma(
            send_ccw.at[N_DEV - 1], recv_ccw.at[N_DEV - 1],
            ccw_send_sems.at[N_DEV - 1], ccw_recv_sems.at[N_DEV - 1], left,
        )
        out_ref[cw_rows(c_own_cw), :] = red_cw
        out_ref[ccw_rows(c_own_ccw), :] = red_ccw

        for h in range(1, N_DEV - 1):
            s = (N_DEV - 1) + h
            r_cw.wait()
            r_cw = start_rdma(
                recv_cw.at[s - 1], recv_cw.at[s],
                cw_send_sems.at[s], cw_recv_sems.at[s], right,
            )
            r_ccw.wait()
            r_ccw = start_rdma(
                recv_ccw.at[s - 1], recv_ccw.at[s],
                ccw_send_sems.at[s], ccw_recv_sems.at[s], left,
            )
            out_ref[cw_rows(cw_chunk(h - 1)), :] = recv_cw[s - 1].astype(f32)
            out_ref[ccw_rows(ccw_chunk(h - 1)), :] = recv_ccw[s - 1].astype(
                f32
            )
        r_cw.wait()
        r_ccw.wait()
        last = 2 * (N_DEV - 1) - 1
        out_ref[cw_rows(cw_chunk(N_DEV - 2)), :] = recv_cw[last].astype(f32)
        out_ref[ccw_rows(ccw_chunk(N_DEV - 2)), :] = recv_ccw[last].astype(
            f32
        )

    n_stage = 2 * (N_DEV - 1)
    out = pl.pallas_call(
        body,
        out_shape=jax.ShapeDtypeStruct((SQ, D_OUT), jnp.float32),
        in_specs=[pl.BlockSpec(memory_space=pltpu.VMEM)] * 5,
        out_specs=pl.BlockSpec(memory_space=pltpu.VMEM),
        scratch_shapes=[
            pltpu.VMEM((SQ, D_OUT), jnp.float32),
            pltpu.VMEM((N_DEV, C, D_OUT), jnp.bfloat16),
            pltpu.VMEM((n_stage, C, D_OUT), jnp.bfloat16),
            pltpu.VMEM((N_DEV, C, D_OUT), jnp.bfloat16),
            pltpu.VMEM((n_stage, C, D_OUT), jnp.bfloat16),
            pltpu.SemaphoreType.DMA((n_stage,)),
            pltpu.SemaphoreType.DMA((n_stage,)),
            pltpu.SemaphoreType.DMA((n_stage,)),
            pltpu.SemaphoreType.DMA((n_stage,)),
        ],
        compiler_params=pltpu.CompilerParams(collective_id=0),
    )(x2, Wq_i, K2, V2, Wo_i)
    return out.reshape(1, SQ, D_OUT)


# device time: 52906 ns/iter; 1.2035x vs baseline; 1.2035x over previous
import jax
import jax.numpy as jnp
from jax import lax
from jax.experimental import pallas as pl
from jax.experimental.pallas import tpu as pltpu

N_DEV = 4
SQ = 1024
SKV = 1024
H_LOC = 8
DH = 128
D_LOC = H_LOC * DH
D_OUT = 1024
WINDOW = 128
SCALE = 0.08838834764831843
C = 128
BLK = 256
KB = BLK + 2 * WINDOW


def kernel(x, Wq, K_ext, V_ext, Wo):
    i = lax.axis_index("i")
    x2 = x.reshape(SQ, x.shape[-1]).astype(jnp.bfloat16)
    Wq_i = lax.dynamic_slice_in_dim(Wq, i * D_LOC, D_LOC, axis=1).astype(
        jnp.bfloat16
    )
    Wo_i = lax.dynamic_slice_in_dim(Wo, i * D_LOC, D_LOC, axis=0).astype(
        jnp.bfloat16
    )
    K2 = K_ext.reshape(SKV, D_LOC).astype(jnp.bfloat16)
    V2 = V_ext.reshape(SKV, D_LOC).astype(jnp.bfloat16)

    def body(
        x_ref,
        wq_ref,
        k_ref,
        v_ref,
        wo_ref,
        out_ref,
        q_ref,
        ctx_ref,
        acc_ref,
        send_cw,
        recv_cw,
        send_ccw,
        recv_ccw,
        cw_send_sems,
        cw_recv_sems,
        ccw_send_sems,
        ccw_recv_sems,
    ):
        my = lax.axis_index("i")
        left = lax.rem(my + (N_DEV - 1), N_DEV)
        right = lax.rem(my + 1, N_DEV)
        f32 = jnp.float32
        bf16 = jnp.bfloat16

        q_ref[...] = jnp.dot(
            x_ref[...], wq_ref[...], preferred_element_type=f32
        ).astype(bf16)

        qi = lax.broadcasted_iota(jnp.int32, (BLK, KB), 0)
        ki = lax.broadcasted_iota(jnp.int32, (BLK, KB), 1)

        def compute_block(j):
            r0 = pl.multiple_of(lax.rem(j + N_DEV, N_DEV) * BLK, BLK)
            kb = pl.multiple_of(
                jnp.clip(r0 - WINDOW, 0, SKV - KB), WINDOW
            )
            mask = jnp.abs((qi + r0) - (ki + kb)) <= WINDOW
            for h in range(H_LOC):
                qh = q_ref[pl.ds(r0, BLK), h * DH : (h + 1) * DH]
                kh = k_ref[pl.ds(kb, KB), h * DH : (h + 1) * DH]
                vh = v_ref[pl.ds(kb, KB), h * DH : (h + 1) * DH]
                s = (
                    lax.dot_general(
                        qh, kh, (((1,), (1,)), ((), ())),
                        preferred_element_type=f32,
                    )
                    * SCALE
                )
                s = jnp.where(mask, s, -1e9)
                m = jnp.max(s, axis=1, keepdims=True)
                w = jnp.exp(s - m)
                w = w / jnp.sum(w, axis=1, keepdims=True)
                ctx_ref[:, h * DH : (h + 1) * DH] = jnp.dot(
                    w.astype(bf16), vh, preferred_element_type=f32
                ).astype(bf16)
            acc_ref[pl.ds(r0, BLK), :] = jnp.dot(
                ctx_ref[...], wo_ref[...], preferred_element_type=f32
            )

        def cw_rows(c):
            return pl.ds(pl.multiple_of(c * BLK, C), C)

        def ccw_rows(c):
            return pl.ds(pl.multiple_of(c * BLK + C, C), C)

        def start_rdma(src, dst, ssem, rsem, dev):
            r = pltpu.make_async_remote_copy(
                src_ref=src,
                dst_ref=dst,
                send_sem=ssem,
                recv_sem=rsem,
                device_id=(dev,),
                device_id_type=pl.DeviceIdType.MESH,
            )
            r.start()
            return r

        compute_block(my)

        barrier_sem = pltpu.get_barrier_semaphore()
        for nbr in [left, right]:
            pl.semaphore_signal(
                barrier_sem,
                inc=1,
                device_id=(nbr,),
                device_id_type=pl.DeviceIdType.MESH,
            )
        pl.semaphore_wait(barrier_sem, 2)

        send_cw[0] = acc_ref[cw_rows(my), :].astype(bf16)
        r_cw = start_rdma(
            send_cw.at[0], recv_cw.at[0],
            cw_send_sems.at[0], cw_recv_sems.at[0], right,
        )
        send_ccw[0] = acc_ref[ccw_rows(my), :].astype(bf16)
        r_ccw = start_rdma(
            send_ccw.at[0], recv_ccw.at[0],
            ccw_send_sems.at[0], ccw_recv_sems.at[0], left,
        )

        for h in range(1, N_DEV - 1):
            compute_block(my - h)
            if h == 1:
                compute_block(my + h)
            c_cw = lax.rem(my - h + 2 * N_DEV, N_DEV)
            c_ccw = lax.rem(my + h, N_DEV)
            r_cw.wait()
            r_ccw.wait()
            send_cw[h] = (
                acc_ref[cw_rows(c_cw), :] + recv_cw[h - 1].astype(f32)
            ).astype(bf16)
            r_cw = start_rdma(
                send_cw.at[h], recv_cw.at[h],
                cw_send_sems.at[h], cw_recv_sems.at[h], right,
            )
            send_ccw[h] = (
                acc_ref[ccw_rows(c_ccw), :] + recv_ccw[h - 1].astype(f32)
            ).astype(bf16)
            r_ccw = start_rdma(
                send_ccw.at[h], recv_ccw.at[h],
                ccw_send_sems.at[h], ccw_recv_sems.at[h], left,
            )

        r_cw.wait()
        r_ccw.wait()

        c_own_cw = lax.rem(my + 1, N_DEV)
        red_cw = (
            acc_ref[cw_rows(c_own_cw), :]
            + recv_cw[N_DEV - 2].astype(f32)
        ).astype(bf16)
        send_cw[N_DEV - 1] = red_cw
        r_cw = start_rdma(
            send_cw.at[N_DEV - 1], recv_cw.at[N_DEV - 1],
            cw_send_sems.at[N_DEV - 1], cw_recv_sems.at[N_DEV - 1], right,
        )
        c_own_ccw = lax.rem(my + N_DEV - 1, N_DEV)
        red_ccw = (
            acc_ref[ccw_rows(c_own_ccw), :]
            + recv_ccw[N_DEV - 2].astype(f32)
        ).astype(bf16)
        send_ccw[N_DEV - 1] = red_ccw
        r_ccw = start_rdma(
            send_ccw.at[N_DEV - 1], recv_ccw.at[N_DEV - 1],
            ccw_send_sems.at[N_DEV - 1], ccw_recv_sems.at[N_DEV - 1], left,
        )
        out_ref[0, cw_rows(c_own_cw), :] = red_cw
        out_ref[0, ccw_rows(c_own_ccw), :] = red_ccw

        for h in range(1, N_DEV - 1):
            s = (N_DEV - 1) + h
            r_cw.wait()
            r_cw = start_rdma(
                recv_cw.at[s - 1], recv_cw.at[s],
                cw_send_sems.at[s], cw_recv_sems.at[s], right,
            )
            r_ccw.wait()
            r_ccw = start_rdma(
                recv_ccw.at[s - 1], recv_ccw.at[s],
                ccw_send_sems.at[s], ccw_recv_sems.at[s], left,
            )
            c_r_cw = lax.rem(my - (h - 1) + 2 * N_DEV, N_DEV)
            c_r_ccw = lax.rem(my + (h - 1), N_DEV)
            out_ref[0, cw_rows(c_r_cw), :] = recv_cw[s - 1]
            out_ref[0, ccw_rows(c_r_ccw), :] = recv_ccw[s - 1]
        r_cw.wait()
        r_ccw.wait()
        last = 2 * (N_DEV - 1) - 1
        out_ref[0, cw_rows(lax.rem(my - (N_DEV - 2) + 2 * N_DEV, N_DEV)), :] = (
            recv_cw[last]
        )
        out_ref[0, ccw_rows(lax.rem(my + (N_DEV - 2), N_DEV)), :] = (
            recv_ccw[last]
        )

    n_stage = 2 * (N_DEV - 1)
    out = pl.pallas_call(
        body,
        out_shape=jax.ShapeDtypeStruct((1, SQ, D_OUT), jnp.bfloat16),
        in_specs=[pl.BlockSpec(memory_space=pltpu.VMEM)] * 5,
        out_specs=pl.BlockSpec(memory_space=pltpu.VMEM),
        scratch_shapes=[
            pltpu.VMEM((SQ, D_LOC), jnp.bfloat16),
            pltpu.VMEM((BLK, D_LOC), jnp.bfloat16),
            pltpu.VMEM((SQ, D_OUT), jnp.float32),
            pltpu.VMEM((N_DEV, C, D_OUT), jnp.bfloat16),
            pltpu.VMEM((n_stage, C, D_OUT), jnp.bfloat16),
            pltpu.VMEM((N_DEV, C, D_OUT), jnp.bfloat16),
            pltpu.VMEM((n_stage, C, D_OUT), jnp.bfloat16),
            pltpu.SemaphoreType.DMA((n_stage,)),
            pltpu.SemaphoreType.DMA((n_stage,)),
            pltpu.SemaphoreType.DMA((n_stage,)),
            pltpu.SemaphoreType.DMA((n_stage,)),
        ],
        compiler_params=pltpu.CompilerParams(collective_id=0),
    )(x2, Wq_i, K2, V2, Wo_i)
    return out
